# Initial kernel scaffold; baseline (speedup 1.0000x reference)
import functools

import jax
import jax.numpy as jnp
from jax import lax
from jax.experimental import pallas as pl
from jax.experimental.pallas import tpu as pltpu
from jax.experimental.pallas import tpu_sc as plsc


def _fold_body(table_ref, w_ref, b_ref, t_ref):
    t_ref[...] = (
        lax.dot_general(
            table_ref[...], w_ref[...],
            dimension_numbers=(((1,), (1,)), ((), ())),
            preferred_element_type=jnp.float32,
        )
        + b_ref[...]
    )


def _fold_table(table, W, b):
    V, E = table.shape
    return pl.pallas_call(
        _fold_body,
        out_shape=jax.ShapeDtypeStruct((V, E), jnp.float32),
    )(table, W, b.reshape(1, E))


_ROWS_PER_ITER = 8
_CHUNKS = ((0, 128), (128, 72))


def _sc_gather(T, x):
    B, L = x.shape
    V, E = T.shape
    info = plsc.get_sparse_core_info()
    NC, NS = info.num_cores, info.num_subcores
    NW = NC * NS
    rows_per_worker = B // NW
    n_iter = rows_per_worker // _ROWS_PER_ITER

    mesh = plsc.VectorSubcoreMesh(core_axis_name="c", subcore_axis_name="s")

    @functools.partial(
        pl.kernel,
        out_type=jax.ShapeDtypeStruct((B, L, E), jnp.float32),
        mesh=mesh,
        scratch_types=[
            pltpu.VMEM((_ROWS_PER_ITER, L), jnp.int32),
            pltpu.VMEM((_ROWS_PER_ITER, L, E), jnp.float32),
            pltpu.SemaphoreType.DMA,
        ],
    )
    def k(t_hbm, x_hbm, out_hbm, idx_v, rows_v, sem):
        wid = lax.axis_index("s") * NC + lax.axis_index("c")
        base = wid * rows_per_worker

        def body(i, _):
            r0 = base + i * _ROWS_PER_ITER
            pltpu.sync_copy(x_hbm.at[pl.ds(r0, _ROWS_PER_ITER)], idx_v)
            copies = []
            for j in range(_ROWS_PER_ITER):
                for off, n in _CHUNKS:
                    copies.append(pltpu.async_copy(
                        t_hbm.at[idx_v.at[j, pl.ds(off, n)]],
                        rows_v.at[j, pl.ds(off, n)],
                        sem,
                    ))
            for c in copies:
                c.wait()
            pltpu.sync_copy(rows_v, out_hbm.at[pl.ds(r0, _ROWS_PER_ITER)])
            return ()

        lax.fori_loop(0, n_iter, body, ())

    return k(T, x)


def kernel(x, table, W, b):
    T = _fold_table(table, W, b)
    return _sc_gather(T, x)


# speedup vs baseline: 3.4169x; 3.4169x over previous
import functools

import jax
import jax.numpy as jnp
from jax import lax
from jax.experimental import pallas as pl
from jax.experimental.pallas import tpu as pltpu
from jax.experimental.pallas import tpu_sc as plsc

_VOCAB = 10
_EMB = 20
_QPR = 50
_KPAD = 56


def _quad_table_body(table_ref, w_ref, b_ref, t4_ref):
    t = (
        lax.dot_general(
            table_ref[...], w_ref[...],
            dimension_numbers=(((1,), (1,)), ((), ())),
            preferred_element_type=jnp.float32,
            precision=lax.Precision.HIGHEST,
        )
        + b_ref[...]
    )
    v = _VOCAB
    left = jnp.broadcast_to(t[:, None, :], (v, v, _EMB)).reshape(v * v, _EMB)
    right = jnp.broadcast_to(t[None, :, :], (v, v, _EMB)).reshape(v * v, _EMB)
    t2 = jnp.concatenate([left, right], axis=1)
    p = v * v
    left4 = jnp.broadcast_to(t2[:, None, :], (p, p, 2 * _EMB)).reshape(p * p, 2 * _EMB)
    right4 = jnp.broadcast_to(t2[None, :, :], (p, p, 2 * _EMB)).reshape(p * p, 2 * _EMB)
    t4_ref[...] = jnp.concatenate([left4, right4], axis=1)


def _quad_table(table, W, b):
    V, E = table.shape
    return pl.pallas_call(
        _quad_table_body,
        out_shape=jax.ShapeDtypeStruct((V**4, 4 * E), jnp.float32),
    )(table, W, b.reshape(1, E))


def _keys_body(x_ref, k_ref):
    bm, L = x_ref.shape
    xf = x_ref[...].astype(jnp.float32)
    d = lax.broadcasted_iota(jnp.int32, (L, _QPR), 0)
    q = lax.broadcasted_iota(jnp.int32, (L, _QPR), 1)
    pa = jnp.where(d == 4 * q, 10.0, 0.0) + jnp.where(d == 4 * q + 1, 1.0, 0.0)
    pb = jnp.where(d == 4 * q + 2, 10.0, 0.0) + jnp.where(d == 4 * q + 3, 1.0, 0.0)
    ka = lax.dot_general(xf, pa, (((1,), (0,)), ((), ())),
                         preferred_element_type=jnp.float32,
                         precision=lax.Precision.HIGHEST)
    kb = lax.dot_general(xf, pb, (((1,), (0,)), ((), ())),
                         preferred_element_type=jnp.float32,
                         precision=lax.Precision.HIGHEST)
    k = ka.astype(jnp.int32) * 100 + kb.astype(jnp.int32)
    k_ref[...] = jnp.concatenate(
        [k, jnp.zeros((bm, _KPAD - _QPR), jnp.int32)], axis=1)


def _quad_keys(x):
    B, L = x.shape
    BM = 512
    return pl.pallas_call(
        _keys_body,
        out_shape=jax.ShapeDtypeStruct((B, _KPAD), jnp.int32),
        grid=(B // BM,),
        in_specs=[pl.BlockSpec((BM, L), lambda i: (i, 0))],
        out_specs=pl.BlockSpec((BM, _KPAD), lambda i: (i, 0)),
    )(x)


_ROWS_PER_ITER = 8


def _sc_gather(T4, keys):
    B = keys.shape[0]
    D = T4.shape[1]
    info = plsc.get_sparse_core_info()
    NC, NS = info.num_cores, info.num_subcores
    NW = NC * NS
    rows_per_worker = B // NW
    n_iter = rows_per_worker // _ROWS_PER_ITER

    mesh = plsc.VectorSubcoreMesh(core_axis_name="c", subcore_axis_name="s")

    @functools.partial(
        pl.kernel,
        out_type=jax.ShapeDtypeStruct((B, _QPR, D), jnp.float32),
        mesh=mesh,
        scratch_types=[
            pltpu.VMEM((_ROWS_PER_ITER, _KPAD), jnp.int32),
            pltpu.VMEM((_ROWS_PER_ITER, _KPAD, D), jnp.float32),
            pltpu.SemaphoreType.DMA,
        ],
        compiler_params=pltpu.CompilerParams(use_tc_tiling_on_sc=False),
    )
    def k(t4_hbm, k_hbm, out_hbm, keys_v, rows_v, sem):
        wid = lax.axis_index("s") * NC + lax.axis_index("c")
        base = wid * rows_per_worker

        def body(i, _):
            r0 = base + i * _ROWS_PER_ITER
            pltpu.sync_copy(k_hbm.at[pl.ds(r0, _ROWS_PER_ITER)], keys_v)
            copies = [
                pltpu.async_copy(
                    t4_hbm.at[keys_v.at[j]], rows_v.at[j], sem)
                for j in range(_ROWS_PER_ITER)
            ]
            for c in copies:
                c.wait()
            for j in range(_ROWS_PER_ITER):
                pltpu.sync_copy(rows_v.at[j, pl.ds(0, _QPR)],
                                out_hbm.at[r0 + j])
            return ()

        lax.fori_loop(0, n_iter, body, ())

    return k(T4, keys)


def kernel(x, table, W, b):
    B, L = x.shape
    T4 = _quad_table(table, W, b)
    keys = _quad_keys(x)
    out = _sc_gather(T4, keys)
    return out.reshape(B, L, _EMB)
